# baseline (speedup 1.0000x reference)
import jax
import jax.numpy as jnp
from jax import lax
from jax.experimental import pallas as pl
from jax.experimental.pallas import tpu as pltpu
from jax.experimental.pallas import tpu_sc as plsc

N = 10000
E = 320000
D = 128
NC = 2
NS = 16
NW = NC * NS
CHUNK = 128
KE = 84
NBUF = 3
NIB = 6
NCHR = E // CHUNK
NCHP = NW * KE - NCHR
NP = 10112
NR = 10040
RPT = NP // NS
_C632 = ((0, 128), (128, 128), (256, 128), (384, 128), (512, 120))
_C608 = ((0, 128), (128, 128), (256, 128), (384, 128), (512, 96))


def _idx_fetch(srcv, dstv, padc, dst_slot, q, isem, dst_only=False):
    @pl.when(q < NCHR)
    def _():
        off = pl.ds(q * CHUNK, CHUNK)
        if not dst_only:
            pltpu.async_copy(srcv.at[off], dst_slot.at[0], isem)
        pltpu.async_copy(dstv.at[off], dst_slot.at[1], isem)

    @pl.when(q >= NCHR)
    def _():
        if dst_only:
            pltpu.async_copy(padc.at[q - NCHR, 1], dst_slot.at[1], isem)
        else:
            pltpu.async_copy(padc.at[q - NCHR], dst_slot, isem)


def _deg_body(srcv, dstv, padc, out_hbm, acc, dst_v, ones_v, bounce,
              isem, dsem):
    core = lax.axis_index("c")
    sub = lax.axis_index("s")
    wid = core * NS + sub

    def fill_ones(i, c):
        ones_v[i, :] = jnp.ones((16,), jnp.float32)
        return c
    lax.fori_loop(0, CHUNK, fill_ones, 0)

    def fill_zero(i, c):
        bounce[i, :] = jnp.zeros((16,), jnp.float32)
        return c
    lax.fori_loop(0, RPT, fill_zero, 0)

    dbase = sub * RPT
    pltpu.sync_copy(bounce, acc.at[pl.ds(dbase, RPT)])

    def stage(j, c):
        _idx_fetch(srcv, dstv, padc, dst_v.at[pl.ds(2 * j, 2)],
                   wid * KE + j, isem, dst_only=True)
        return c
    lax.fori_loop(0, KE, stage, 0)

    def drain_idx(j, c):
        pltpu.make_async_copy(padc.at[0, 1], dst_v.at[0], isem).wait()
        return c
    lax.fori_loop(0, KE, drain_idx, 0)
    plsc.subcore_barrier()

    def round_body(r, c):
        for b in range(6):
            j = r * 6 + b
            pltpu.async_copy(ones_v, acc.at[dst_v.at[2 * j + 1]], dsem,
                             add=True)
        for b in range(6):
            pltpu.make_async_copy(ones_v, acc.at[dst_v.at[1]], dsem).wait()
        return c
    lax.fori_loop(0, KE // 6, round_body, 0)

    plsc.subcore_barrier()
    pltpu.sync_copy(acc.at[pl.ds(dbase, RPT)], bounce)
    pltpu.sync_copy(bounce, out_hbm.at[core, pl.ds(dbase, RPT)])


_deg = pl.kernel(
    _deg_body,
    out_type=jax.ShapeDtypeStruct((NC, NP, 16), jnp.float32),
    mesh=plsc.VectorSubcoreMesh(core_axis_name="c", subcore_axis_name="s"),
    scratch_types=[
        pltpu.VMEM_SHARED((NP, 16), jnp.float32),
        pltpu.VMEM((2 * KE, CHUNK), jnp.int32),
        pltpu.VMEM((CHUNK, 16), jnp.float32),
        pltpu.VMEM((RPT, 16), jnp.float32),
        pltpu.SemaphoreType.DMA,
        pltpu.SemaphoreType.DMA,
    ],
)


def _edge_body(s_hbm, srcv, dstv, padc, out_hbm,
               acc, ib, buf, g0, g1, g2, s0, s1, s2, isem):
    core = lax.axis_index("c")
    sub = lax.axis_index("s")
    wid = core * NS + sub
    qbase = wid * KE
    gsems = (g0, g1, g2)
    ssems = (s0, s1, s2)

    def initio(chunks, base, write):
        for off, sz in chunks:
            rs = pl.ds(base + off, sz)
            bs = buf.at[0, pl.ds(0, sz)]
            if write:
                pltpu.sync_copy(acc.at[rs], bs)
                pltpu.sync_copy(bs, out_hbm.at[core, rs])
            else:
                pltpu.sync_copy(s_hbm.at[rs], bs)
                pltpu.sync_copy(bs, acc.at[rs])

    def tile_io(write):
        @pl.when(sub < 13)
        def _():
            initio(_C632, sub * 632, write)

        @pl.when(sub >= 13)
        def _():
            initio(_C608, 8216 + (sub - 13) * 608, write)

    tile_io(False)

    for q in range(NIB):
        _idx_fetch(srcv, dstv, padc, ib.at[q], qbase + q, isem)

    def wait_idx():
        pltpu.make_async_copy(padc.at[0], ib.at[0], isem).wait()

    def gather_start(slot, b):
        pltpu.async_copy(s_hbm.at[ib.at[slot, 0]], buf.at[b], gsems[b])

    def gather_wait(slot, b):
        pltpu.make_async_copy(s_hbm.at[ib.at[slot, 0]], buf.at[b],
                              gsems[b]).wait()

    def scat_start(slot, b):
        pltpu.async_copy(buf.at[b], acc.at[ib.at[slot, 1]], ssems[b],
                         add=True)

    def scat_wait(slot, b):
        pltpu.make_async_copy(buf.at[b], acc.at[ib.at[slot, 1]],
                              ssems[b]).wait()

    plsc.subcore_barrier()

    wait_idx()
    wait_idx()
    gather_start(0, 0)
    gather_start(1, 1)

    NRND = KE // NIB

    def round_body(o, c):
        for b in range(NIB):
            bb = b % NBUF
            pb = (b - 1) % NBUF
            gather_wait(b, bb)
            scat_start(b, bb)
            if b == 0:
                @pl.when(o > 0)
                def _():
                    scat_wait((b - 1) % NIB, pb)
            else:
                scat_wait(b - 1, pb)
            if b < 4:
                wait_idx()
                gather_start((b + 2) % NIB, pb)
            else:
                @pl.when(o < NRND - 1)
                def _():
                    wait_idx()
                    gather_start((b + 2) % NIB, pb)
            if b == 0:
                @pl.when(o > 0)
                def _():
                    _idx_fetch(srcv, dstv, padc, ib.at[(b - 1) % NIB],
                               qbase + o * NIB + b + 5, isem)
            else:
                @pl.when(o < NRND - 1)
                def _():
                    _idx_fetch(srcv, dstv, padc, ib.at[b - 1],
                               qbase + o * NIB + b + 5, isem)
        return c

    lax.fori_loop(0, NRND, round_body, 0)
    scat_wait(NIB - 1, (KE - 1) % NBUF)
    plsc.subcore_barrier()

    tile_io(True)


_edge = pl.kernel(
    _edge_body,
    out_type=jax.ShapeDtypeStruct((NC, NR, D), jnp.float32),
    mesh=plsc.VectorSubcoreMesh(core_axis_name="c", subcore_axis_name="s"),
    scratch_types=[
        pltpu.VMEM_SHARED((NR, D), jnp.float32),
        pltpu.VMEM((NIB, 2, CHUNK), jnp.int32),
        pltpu.VMEM((NBUF, CHUNK, D), jnp.float32),
    ] + [pltpu.SemaphoreType.DMA] * 7,
)


def _dis_from_counts(cnt_ref):
    c = cnt_ref[0, :, 0:1] + cnt_ref[1, :, 0:1] + 1.0
    return lax.rsqrt(c)


def _mm1_body(x_ref, w_ref, o_ref):
    o_ref[0:N, :] = jnp.dot(x_ref[...], w_ref[...],
                            preferred_element_type=jnp.float32)
    o_ref[N:NP, :] = jnp.zeros((NP - N, D), jnp.float32)


def _sc1_body(u_ref, cnt_ref, o_ref):
    o_ref[...] = u_ref[...] * _dis_from_counts(cnt_ref)


def _tc2_body(acc_ref, s_ref, cnt_ref, b_ref, w_ref, o_ref):
    dis = _dis_from_counts(cnt_ref)
    g = acc_ref[0, 0:N, :] + acc_ref[1, 0:N, :] - s_ref[0:N, :]
    gp = jnp.concatenate(
        [g, jnp.zeros((NP - N, D), jnp.float32)], axis=0)
    h = jnp.maximum(gp * dis + b_ref[...], 0.0)
    o_ref[...] = jnp.dot(h, w_ref[...],
                         preferred_element_type=jnp.float32) * dis


def _tc3_body(acc_ref, s_ref, cnt_ref, b_ref, o_ref):
    c = cnt_ref[0, 0:N, 0:1] + cnt_ref[1, 0:N, 0:1] + 1.0
    dis = lax.rsqrt(c)
    g = acc_ref[0, 0:N, :] + acc_ref[1, 0:N, :] - s_ref[0:N, :]
    o_ref[...] = g * dis + b_ref[...]


_mm1 = pl.pallas_call(
    _mm1_body, out_shape=jax.ShapeDtypeStruct((NP, D), jnp.float32))
_sc1 = pl.pallas_call(
    _sc1_body, out_shape=jax.ShapeDtypeStruct((NP, D), jnp.float32))
_tc2 = pl.pallas_call(
    _tc2_body, out_shape=jax.ShapeDtypeStruct((NP, D), jnp.float32))
_tc3 = pl.pallas_call(
    _tc3_body, out_shape=jax.ShapeDtypeStruct((N, D), jnp.float32))


def kernel(x, edge_index, W1, b1, W2, b2):
    ei = edge_index.astype(jnp.int32)
    srcv = ei[0]
    dstv = ei[1]
    padr = N + jnp.arange(NCHP * CHUNK, dtype=jnp.int32) % (NR - N)
    padc = jnp.broadcast_to(padr.reshape(NCHP, 1, CHUNK),
                            (NCHP, 2, CHUNK))
    b1r = b1.reshape(1, D)
    b2r = b2.reshape(1, D)

    cnt = _deg(srcv, dstv, padc)
    u1 = _mm1(x, W1)
    s1 = _sc1(u1, cnt)
    a1 = _edge(s1, srcv, dstv, padc)
    s2 = _tc2(a1, s1, cnt, b1r, W2)
    a2 = _edge(s2, srcv, dstv, padc)
    return _tc3(a2, s2, cnt, b2r)


# speedup vs baseline: 1.0538x; 1.0538x over previous
import jax
import jax.numpy as jnp
import numpy as np
from jax import lax
from jax.experimental import pallas as pl
from jax.experimental.pallas import tpu as pltpu
from jax.experimental.pallas import tpu_sc as plsc

N = 10000
E = 320000
D = 128
NC = 2
NS = 16
NW = NC * NS
CHUNK = 128
KE = 84
NBUF = 3
NIB = 6
NCHR = E // CHUNK
NCHP = NW * KE - NCHR
NP = 10112
NR = 10040
RPT = NP // NS
_C632 = ((0, 128), (128, 128), (256, 128), (384, 128), (512, 120))
_C608 = ((0, 128), (128, 128), (256, 128), (384, 128), (512, 96))


def _idx_fetch(srcv, dstv, padc, dst_slot, q, isem, dst_only=False):
    @pl.when(q < NCHR)
    def _():
        off = pl.ds(q * CHUNK, CHUNK)
        if not dst_only:
            pltpu.async_copy(srcv.at[off], dst_slot.at[0], isem)
        pltpu.async_copy(dstv.at[off], dst_slot.at[1], isem)

    @pl.when(q >= NCHR)
    def _():
        if dst_only:
            pltpu.async_copy(padc.at[q - NCHR, 1], dst_slot.at[1], isem)
        else:
            pltpu.async_copy(padc.at[q - NCHR], dst_slot, isem)


def _deg_body(srcv, dstv, padc, out_hbm, acc, dst_v, ones_v, bounce,
              isem, dsem):
    core = lax.axis_index("c")
    sub = lax.axis_index("s")
    wid = core * NS + sub

    def fill_ones(i, c):
        ones_v[i, :] = jnp.ones((16,), jnp.float32)
        return c
    lax.fori_loop(0, CHUNK, fill_ones, 0)

    def fill_zero(i, c):
        bounce[i, :] = jnp.zeros((16,), jnp.float32)
        return c
    lax.fori_loop(0, RPT, fill_zero, 0)

    dbase = sub * RPT
    pltpu.sync_copy(bounce, acc.at[pl.ds(dbase, RPT)])

    def stage(j, c):
        _idx_fetch(srcv, dstv, padc, dst_v.at[j],
                   wid * KE + j, isem, dst_only=True)
        return c
    lax.fori_loop(0, KE, stage, 0)

    def drain_idx(j, c):
        pltpu.make_async_copy(padc.at[0, 1], dst_v.at[0, 1], isem).wait()
        return c
    lax.fori_loop(0, KE, drain_idx, 0)
    plsc.subcore_barrier()

    def round_body(r, c):
        for b in range(6):
            j = r * 6 + b
            pltpu.async_copy(ones_v, acc.at[dst_v.at[j, 1]], dsem,
                             add=True)
        for b in range(6):
            pltpu.make_async_copy(ones_v, acc.at[dst_v.at[0, 1]],
                                  dsem).wait()
        return c
    lax.fori_loop(0, KE // 6, round_body, 0)

    plsc.subcore_barrier()
    pltpu.sync_copy(acc.at[pl.ds(dbase, RPT)], bounce)
    pltpu.sync_copy(bounce, out_hbm.at[core, pl.ds(dbase, RPT)])


_deg = pl.kernel(
    _deg_body,
    out_type=jax.ShapeDtypeStruct((NC, NP, 16), jnp.float32),
    mesh=plsc.VectorSubcoreMesh(core_axis_name="c", subcore_axis_name="s"),
    scratch_types=[
        pltpu.VMEM_SHARED((NP, 16), jnp.float32),
        pltpu.VMEM((KE, 2, CHUNK), jnp.int32),
        pltpu.VMEM((CHUNK, 16), jnp.float32),
        pltpu.VMEM((RPT, 16), jnp.float32),
        pltpu.SemaphoreType.DMA,
        pltpu.SemaphoreType.DMA,
    ],
)


def _edge_body(s_hbm, srcv, dstv, padc, out_hbm,
               acc, ib, buf, g0, g1, g2, s0, s1, s2, isem):
    core = lax.axis_index("c")
    sub = lax.axis_index("s")
    wid = core * NS + sub
    qbase = wid * KE
    gsems = (g0, g1, g2)
    ssems = (s0, s1, s2)

    def initio(chunks, base, write):
        for off, sz in chunks:
            rs = pl.ds(base + off, sz)
            bs = buf.at[0, pl.ds(0, sz)]
            if write:
                pltpu.sync_copy(acc.at[rs], bs)
                pltpu.sync_copy(bs, out_hbm.at[core, rs])
            else:
                pltpu.sync_copy(s_hbm.at[rs], bs)
                pltpu.sync_copy(bs, acc.at[rs])

    def tile_io(write):
        @pl.when(sub < 13)
        def _():
            initio(_C632, sub * 632, write)

        @pl.when(sub >= 13)
        def _():
            initio(_C608, 8216 + (sub - 13) * 608, write)

    tile_io(False)

    for q in range(NIB):
        _idx_fetch(srcv, dstv, padc, ib.at[q], qbase + q, isem)

    def wait_idx():
        pltpu.make_async_copy(padc.at[0], ib.at[0], isem).wait()

    def gather_start(slot, b):
        pltpu.async_copy(s_hbm.at[ib.at[slot, 0]], buf.at[b], gsems[b])

    def gather_wait(slot, b):
        pltpu.make_async_copy(s_hbm.at[ib.at[slot, 0]], buf.at[b],
                              gsems[b]).wait()

    def scat_start(slot, b):
        pltpu.async_copy(buf.at[b], acc.at[ib.at[slot, 1]], ssems[b],
                         add=True)

    def scat_wait(slot, b):
        pltpu.make_async_copy(buf.at[b], acc.at[ib.at[slot, 1]],
                              ssems[b]).wait()

    plsc.subcore_barrier()

    wait_idx()
    wait_idx()
    gather_start(0, 0)
    gather_start(1, 1)

    NRND = KE // NIB

    def round_body(o, c):
        for b in range(NIB):
            bb = b % NBUF
            pb = (b - 1) % NBUF
            gather_wait(b, bb)
            scat_start(b, bb)
            if b == 0:
                @pl.when(o > 0)
                def _():
                    scat_wait((b - 1) % NIB, pb)
            else:
                scat_wait(b - 1, pb)
            if b < 4:
                wait_idx()
                gather_start((b + 2) % NIB, pb)
            else:
                @pl.when(o < NRND - 1)
                def _():
                    wait_idx()
                    gather_start((b + 2) % NIB, pb)
            if b == 0:
                @pl.when(o > 0)
                def _():
                    _idx_fetch(srcv, dstv, padc, ib.at[(b - 1) % NIB],
                               qbase + o * NIB + b + 5, isem)
            else:
                @pl.when(o < NRND - 1)
                def _():
                    _idx_fetch(srcv, dstv, padc, ib.at[b - 1],
                               qbase + o * NIB + b + 5, isem)
        return c

    lax.fori_loop(0, NRND, round_body, 0)
    scat_wait(NIB - 1, (KE - 1) % NBUF)
    plsc.subcore_barrier()

    tile_io(True)


_edge = pl.kernel(
    _edge_body,
    out_type=jax.ShapeDtypeStruct((NC, NR, D), jnp.float32),
    mesh=plsc.VectorSubcoreMesh(core_axis_name="c", subcore_axis_name="s"),
    scratch_types=[
        pltpu.VMEM_SHARED((NR, D), jnp.float32),
        pltpu.VMEM((NIB, 2, CHUNK), jnp.int32),
        pltpu.VMEM((NBUF, CHUNK, D), jnp.float32),
    ] + [pltpu.SemaphoreType.DMA] * 7,
)


def _dis_from_counts(cnt_ref):
    c = cnt_ref[0, :, 0:1] + cnt_ref[1, :, 0:1] + 1.0
    return lax.rsqrt(c)


def _mm1_body(x_ref, w_ref, o_ref):
    o_ref[0:N, :] = jnp.dot(x_ref[...], w_ref[...],
                            preferred_element_type=jnp.float32)
    o_ref[N:NP, :] = jnp.zeros((NP - N, D), jnp.float32)


def _sc1_body(u_ref, cnt_ref, o_ref):
    o_ref[...] = u_ref[...] * _dis_from_counts(cnt_ref)


def _tc2_body(acc_ref, s_ref, cnt_ref, b_ref, w_ref, o_ref):
    dis = _dis_from_counts(cnt_ref)
    g = acc_ref[0, 0:N, :] + acc_ref[1, 0:N, :] - s_ref[0:N, :]
    gp = jnp.concatenate(
        [g, jnp.zeros((NP - N, D), jnp.float32)], axis=0)
    h = jnp.maximum(gp * dis + b_ref[...], 0.0)
    o_ref[...] = jnp.dot(h, w_ref[...],
                         preferred_element_type=jnp.float32) * dis


def _tc3_body(acc_ref, s_ref, cnt_ref, b_ref, o_ref):
    c = cnt_ref[0, 0:N, 0:1] + cnt_ref[1, 0:N, 0:1] + 1.0
    dis = lax.rsqrt(c)
    g = acc_ref[0, 0:N, :] + acc_ref[1, 0:N, :] - s_ref[0:N, :]
    o_ref[...] = g * dis + b_ref[...]


_mm1 = pl.pallas_call(
    _mm1_body, out_shape=jax.ShapeDtypeStruct((NP, D), jnp.float32))
_sc1 = pl.pallas_call(
    _sc1_body, out_shape=jax.ShapeDtypeStruct((NP, D), jnp.float32))
_tc2 = pl.pallas_call(
    _tc2_body, out_shape=jax.ShapeDtypeStruct((NP, D), jnp.float32))
_tc3 = pl.pallas_call(
    _tc3_body, out_shape=jax.ShapeDtypeStruct((N, D), jnp.float32))


def kernel(x, edge_index, W1, b1, W2, b2):
    ei = edge_index.astype(jnp.int32)
    srcv = ei[0]
    dstv = ei[1]
    padr = (N + np.arange(NCHP * CHUNK, dtype=np.int32) % (NR - N))
    padc = jnp.asarray(np.broadcast_to(padr.reshape(NCHP, 1, CHUNK),
                                       (NCHP, 2, CHUNK)))
    b1r = b1.reshape(1, D)
    b2r = b2.reshape(1, D)

    cnt = _deg(srcv, dstv, padc)
    u1 = _mm1(x, W1)
    s1 = _sc1(u1, cnt)
    a1 = _edge(s1, srcv, dstv, padc)
    s2 = _tc2(a1, s1, cnt, b1r, W2)
    a2 = _edge(s2, srcv, dstv, padc)
    return _tc3(a2, s2, cnt, b2r)


# speedup vs baseline: 1.1107x; 1.0540x over previous
import jax
import jax.numpy as jnp
import numpy as np
from jax import lax
from jax.experimental import pallas as pl
from jax.experimental.pallas import tpu as pltpu
from jax.experimental.pallas import tpu_sc as plsc

N = 10000
E = 320000
D = 128
NC = 2
NS = 16
NW = NC * NS
CHUNK = 128
KE = 84
NBUF = 3
NIB = 6
NCHR = E // CHUNK
NCHP = NW * KE - NCHR
NP = 10112
NR = 10040
RPT = NP // NS
_C632 = ((0, 128), (128, 128), (256, 128), (384, 128), (512, 120))
_C608 = ((0, 128), (128, 128), (256, 128), (384, 128), (512, 96))


def _idx_fetch(ei, padc, dst_slot, q, isem, dst_only=False):
    @pl.when(q < NCHR)
    def _():
        off = pl.ds(q * CHUNK, CHUNK)
        if not dst_only:
            pltpu.async_copy(ei.at[0, off], dst_slot.at[0], isem)
        pltpu.async_copy(ei.at[1, off], dst_slot.at[1], isem)

    @pl.when(q >= NCHR)
    def _():
        if dst_only:
            pltpu.async_copy(padc.at[q - NCHR, 1], dst_slot.at[1], isem)
        else:
            pltpu.async_copy(padc.at[q - NCHR], dst_slot, isem)


def _deg_body(ei, padc, out_hbm, acc, dst_v, ones_v, bounce,
              isem, dsem):
    core = lax.axis_index("c")
    sub = lax.axis_index("s")
    wid = core * NS + sub

    def fill_ones(i, c):
        ones_v[i, :] = jnp.ones((16,), jnp.float32)
        return c
    lax.fori_loop(0, CHUNK, fill_ones, 0)

    def fill_zero(i, c):
        bounce[i, :] = jnp.zeros((16,), jnp.float32)
        return c
    lax.fori_loop(0, RPT, fill_zero, 0)

    dbase = sub * RPT
    pltpu.sync_copy(bounce, acc.at[pl.ds(dbase, RPT)])

    def stage(j, c):
        _idx_fetch(ei, padc, dst_v.at[j],
                   wid * KE + j, isem, dst_only=True)
        return c
    lax.fori_loop(0, KE, stage, 0)

    def drain_idx(j, c):
        pltpu.make_async_copy(padc.at[0, 1], dst_v.at[0, 1], isem).wait()
        return c
    lax.fori_loop(0, KE, drain_idx, 0)
    plsc.subcore_barrier()

    def round_body(r, c):
        for b in range(6):
            j = r * 6 + b
            pltpu.async_copy(ones_v, acc.at[dst_v.at[j, 1]], dsem,
                             add=True)
        for b in range(6):
            pltpu.make_async_copy(ones_v, acc.at[dst_v.at[0, 1]],
                                  dsem).wait()
        return c
    lax.fori_loop(0, KE // 6, round_body, 0)

    plsc.subcore_barrier()
    pltpu.sync_copy(acc.at[pl.ds(dbase, RPT)], bounce)
    pltpu.sync_copy(bounce, out_hbm.at[core, pl.ds(dbase, RPT)])


_deg = pl.kernel(
    _deg_body,
    out_type=jax.ShapeDtypeStruct((NC, NP, 16), jnp.float32),
    mesh=plsc.VectorSubcoreMesh(core_axis_name="c", subcore_axis_name="s"),
    scratch_types=[
        pltpu.VMEM_SHARED((NP, 16), jnp.float32),
        pltpu.VMEM((KE, 2, CHUNK), jnp.int32),
        pltpu.VMEM((CHUNK, 16), jnp.float32),
        pltpu.VMEM((RPT, 16), jnp.float32),
        pltpu.SemaphoreType.DMA,
        pltpu.SemaphoreType.DMA,
    ],
)


def _edge_body(s_hbm, ei, padc, out_hbm,
               acc, ib, buf, g0, g1, g2, s0, s1, s2, isem):
    core = lax.axis_index("c")
    sub = lax.axis_index("s")
    wid = core * NS + sub
    qbase = wid * KE
    gsems = (g0, g1, g2)
    ssems = (s0, s1, s2)

    def initio(chunks, base, write):
        for off, sz in chunks:
            rs = pl.ds(base + off, sz)
            bs = buf.at[0, pl.ds(0, sz)]
            if write:
                pltpu.sync_copy(acc.at[rs], bs)
                pltpu.sync_copy(bs, out_hbm.at[core, rs])
            else:
                pltpu.sync_copy(s_hbm.at[rs], bs)
                pltpu.sync_copy(bs, acc.at[rs])

    def tile_io(write):
        @pl.when(sub < 13)
        def _():
            initio(_C632, sub * 632, write)

        @pl.when(sub >= 13)
        def _():
            initio(_C608, 8216 + (sub - 13) * 608, write)

    tile_io(False)

    for q in range(NIB):
        _idx_fetch(ei, padc, ib.at[q], qbase + q, isem)

    def wait_idx():
        pltpu.make_async_copy(padc.at[0], ib.at[0], isem).wait()

    def gather_start(slot, b):
        pltpu.async_copy(s_hbm.at[ib.at[slot, 0]], buf.at[b], gsems[b])

    def gather_wait(slot, b):
        pltpu.make_async_copy(s_hbm.at[ib.at[slot, 0]], buf.at[b],
                              gsems[b]).wait()

    def scat_start(slot, b):
        pltpu.async_copy(buf.at[b], acc.at[ib.at[slot, 1]], ssems[b],
                         add=True)

    def scat_wait(slot, b):
        pltpu.make_async_copy(buf.at[b], acc.at[ib.at[slot, 1]],
                              ssems[b]).wait()

    plsc.subcore_barrier()

    wait_idx()
    wait_idx()
    gather_start(0, 0)
    gather_start(1, 1)

    NRND = KE // NIB

    def round_body(o, c):
        for b in range(NIB):
            bb = b % NBUF
            pb = (b - 1) % NBUF
            gather_wait(b, bb)
            scat_start(b, bb)
            if b == 0:
                @pl.when(o > 0)
                def _():
                    scat_wait((b - 1) % NIB, pb)
            else:
                scat_wait(b - 1, pb)
            if b < 4:
                wait_idx()
                gather_start((b + 2) % NIB, pb)
            else:
                @pl.when(o < NRND - 1)
                def _():
                    wait_idx()
                    gather_start((b + 2) % NIB, pb)
            if b == 0:
                @pl.when(o > 0)
                def _():
                    _idx_fetch(ei, padc, ib.at[(b - 1) % NIB],
                               qbase + o * NIB + b + 5, isem)
            else:
                @pl.when(o < NRND - 1)
                def _():
                    _idx_fetch(ei, padc, ib.at[b - 1],
                               qbase + o * NIB + b + 5, isem)
        return c

    lax.fori_loop(0, NRND, round_body, 0)
    scat_wait(NIB - 1, (KE - 1) % NBUF)
    plsc.subcore_barrier()

    tile_io(True)


_edge = pl.kernel(
    _edge_body,
    out_type=jax.ShapeDtypeStruct((NC, NR, D), jnp.float32),
    mesh=plsc.VectorSubcoreMesh(core_axis_name="c", subcore_axis_name="s"),
    scratch_types=[
        pltpu.VMEM_SHARED((NR, D), jnp.float32),
        pltpu.VMEM((NIB, 2, CHUNK), jnp.int32),
        pltpu.VMEM((NBUF, CHUNK, D), jnp.float32),
    ] + [pltpu.SemaphoreType.DMA] * 7,
)


def _dis_from_counts(cnt_ref):
    c = cnt_ref[0, :, 0:1] + cnt_ref[1, :, 0:1] + 1.0
    return lax.rsqrt(c)


def _mm1_body(x_ref, w_ref, o_ref):
    o_ref[0:N, :] = jnp.dot(x_ref[...], w_ref[...],
                            preferred_element_type=jnp.float32)
    o_ref[N:NP, :] = jnp.zeros((NP - N, D), jnp.float32)


def _sc1_body(u_ref, cnt_ref, o_ref):
    o_ref[...] = u_ref[...] * _dis_from_counts(cnt_ref)


def _tc2_body(acc_ref, s_ref, cnt_ref, b_ref, w_ref, o_ref):
    dis = _dis_from_counts(cnt_ref)
    g = acc_ref[0, 0:N, :] + acc_ref[1, 0:N, :] - s_ref[0:N, :]
    gp = jnp.concatenate(
        [g, jnp.zeros((NP - N, D), jnp.float32)], axis=0)
    h = jnp.maximum(gp * dis + b_ref[...], 0.0)
    o_ref[...] = jnp.dot(h, w_ref[...],
                         preferred_element_type=jnp.float32) * dis


def _tc3_body(acc_ref, s_ref, cnt_ref, b_ref, o_ref):
    c = cnt_ref[0, 0:N, 0:1] + cnt_ref[1, 0:N, 0:1] + 1.0
    dis = lax.rsqrt(c)
    g = acc_ref[0, 0:N, :] + acc_ref[1, 0:N, :] - s_ref[0:N, :]
    o_ref[...] = g * dis + b_ref[...]


_mm1 = pl.pallas_call(
    _mm1_body, out_shape=jax.ShapeDtypeStruct((NP, D), jnp.float32))
_sc1 = pl.pallas_call(
    _sc1_body, out_shape=jax.ShapeDtypeStruct((NP, D), jnp.float32))
_tc2 = pl.pallas_call(
    _tc2_body, out_shape=jax.ShapeDtypeStruct((NP, D), jnp.float32))
_tc3 = pl.pallas_call(
    _tc3_body, out_shape=jax.ShapeDtypeStruct((N, D), jnp.float32))


def kernel(x, edge_index, W1, b1, W2, b2):
    ei = edge_index.astype(jnp.int32)
    padr = (N + np.arange(NCHP * CHUNK, dtype=np.int32) % (NR - N))
    padc = jnp.asarray(np.broadcast_to(padr.reshape(NCHP, 1, CHUNK),
                                       (NCHP, 2, CHUNK)))
    b1r = b1.reshape(1, D)
    b2r = b2.reshape(1, D)

    cnt = _deg(ei, padc)
    u1 = _mm1(x, W1)
    s1 = _sc1(u1, cnt)
    a1 = _edge(s1, ei, padc)
    s2 = _tc2(a1, s1, cnt, b1r, W2)
    a2 = _edge(s2, ei, padc)
    return _tc3(a2, s2, cnt, b2r)


# speedup vs baseline: 1.1558x; 1.0406x over previous
import jax
import jax.numpy as jnp
import numpy as np
from jax import lax
from jax.experimental import pallas as pl
from jax.experimental.pallas import tpu as pltpu
from jax.experimental.pallas import tpu_sc as plsc

N = 10000
E = 320000
D = 128
NC = 2
NS = 16
NW = NC * NS
CHUNK = 128
KE = 84
NBUF = 3
NIB = 6
NCHR = E // CHUNK
NCHP = NW * KE - NCHR
NP = 10112
NR = 10040
RPT = NP // NS
_C632 = ((0, 128), (128, 128), (256, 128), (384, 128), (512, 120))
_C608 = ((0, 128), (128, 128), (256, 128), (384, 128), (512, 96))


def _idx_fetch(ei, padc, dst_slot, q, isem, dst_only=False):
    @pl.when(q < NCHR)
    def _():
        off = pl.ds(q * CHUNK, CHUNK)
        if not dst_only:
            pltpu.async_copy(ei.at[0, off], dst_slot.at[0], isem)
        pltpu.async_copy(ei.at[1, off], dst_slot.at[1], isem)

    @pl.when(q >= NCHR)
    def _():
        if dst_only:
            pltpu.async_copy(padc.at[q - NCHR, 1], dst_slot.at[1], isem)
        else:
            pltpu.async_copy(padc.at[q - NCHR], dst_slot, isem)


def _deg_body(ei, padc, out_hbm, acc, dst_v, ones_v, bounce,
              isem, dsem):
    core = lax.axis_index("c")
    sub = lax.axis_index("s")
    wid = core * NS + sub

    def fill_ones(i, c):
        ones_v[i, :] = jnp.ones((16,), jnp.float32)
        return c
    lax.fori_loop(0, CHUNK, fill_ones, 0)

    def fill_zero(i, c):
        bounce[i, :] = jnp.zeros((16,), jnp.float32)
        return c
    lax.fori_loop(0, RPT, fill_zero, 0)

    dbase = sub * RPT
    pltpu.sync_copy(bounce, acc.at[pl.ds(dbase, RPT)])

    def stage(j, c):
        _idx_fetch(ei, padc, dst_v.at[j],
                   wid * KE + j, isem, dst_only=True)
        return c
    lax.fori_loop(0, KE, stage, 0)

    def drain_idx(j, c):
        pltpu.make_async_copy(padc.at[0, 1], dst_v.at[0, 1], isem).wait()
        return c
    lax.fori_loop(0, KE, drain_idx, 0)
    plsc.subcore_barrier()

    def round_body(r, c):
        for b in range(6):
            j = r * 6 + b
            pltpu.async_copy(ones_v, acc.at[dst_v.at[j, 1]], dsem,
                             add=True)
        for b in range(6):
            pltpu.make_async_copy(ones_v, acc.at[dst_v.at[0, 1]],
                                  dsem).wait()
        return c
    lax.fori_loop(0, KE // 6, round_body, 0)

    plsc.subcore_barrier()
    pltpu.sync_copy(acc.at[pl.ds(dbase, RPT)], bounce)
    pltpu.sync_copy(bounce, out_hbm.at[core, pl.ds(dbase, RPT)])


_deg = pl.kernel(
    _deg_body,
    out_type=jax.ShapeDtypeStruct((NC, NP, 16), jnp.float32),
    mesh=plsc.VectorSubcoreMesh(core_axis_name="c", subcore_axis_name="s"),
    scratch_types=[
        pltpu.VMEM_SHARED((NP, 16), jnp.float32),
        pltpu.VMEM((KE, 2, CHUNK), jnp.int32),
        pltpu.VMEM((CHUNK, 16), jnp.float32),
        pltpu.VMEM((RPT, 16), jnp.float32),
        pltpu.SemaphoreType.DMA,
        pltpu.SemaphoreType.DMA,
    ],
)


def _edge_body(s_hbm, ei, padc, out_hbm,
               acc, ib, buf, g0, g1, g2, s0, s1, s2, isem):
    core = lax.axis_index("c")
    sub = lax.axis_index("s")
    wid = core * NS + sub
    qbase = wid * KE
    gsems = (g0, g1, g2)
    ssems = (s0, s1, s2)

    def initio(chunks, base, write):
        for off, sz in chunks:
            rs = pl.ds(base + off, sz)
            bs = buf.at[0, pl.ds(0, sz)]
            if write:
                pltpu.sync_copy(acc.at[rs], bs)
                pltpu.sync_copy(bs, out_hbm.at[core, rs])
            else:
                pltpu.sync_copy(bs, acc.at[rs])

    def tile_io(write):
        @pl.when(sub < 13)
        def _():
            initio(_C632, sub * 632, write)

        @pl.when(sub >= 13)
        def _():
            initio(_C608, 8216 + (sub - 13) * 608, write)

    def zrow(r, c):
        for c8 in range(8):
            buf[0, r, pl.ds(c8 * 16, 16)] = jnp.zeros((16,), jnp.float32)
        return c
    lax.fori_loop(0, CHUNK, zrow, 0)
    tile_io(False)

    for q in range(NIB):
        _idx_fetch(ei, padc, ib.at[q], qbase + q, isem)

    def wait_idx():
        pltpu.make_async_copy(padc.at[0], ib.at[0], isem).wait()

    def gather_start(slot, b):
        pltpu.async_copy(s_hbm.at[ib.at[slot, 0]], buf.at[b], gsems[b])

    def gather_wait(slot, b):
        pltpu.make_async_copy(s_hbm.at[ib.at[slot, 0]], buf.at[b],
                              gsems[b]).wait()

    def scat_start(slot, b):
        pltpu.async_copy(buf.at[b], acc.at[ib.at[slot, 1]], ssems[b],
                         add=True)

    def scat_wait(slot, b):
        pltpu.make_async_copy(buf.at[b], acc.at[ib.at[slot, 1]],
                              ssems[b]).wait()

    plsc.subcore_barrier()

    wait_idx()
    wait_idx()
    gather_start(0, 0)
    gather_start(1, 1)

    NRND = KE // NIB

    def round_body(o, c):
        for b in range(NIB):
            bb = b % NBUF
            pb = (b - 1) % NBUF
            gather_wait(b, bb)
            scat_start(b, bb)
            if b == 0:
                @pl.when(o > 0)
                def _():
                    scat_wait((b - 1) % NIB, pb)
            else:
                scat_wait(b - 1, pb)
            if b < 4:
                wait_idx()
                gather_start((b + 2) % NIB, pb)
            else:
                @pl.when(o < NRND - 1)
                def _():
                    wait_idx()
                    gather_start((b + 2) % NIB, pb)
            if b == 0:
                @pl.when(o > 0)
                def _():
                    _idx_fetch(ei, padc, ib.at[(b - 1) % NIB],
                               qbase + o * NIB + b + 5, isem)
            else:
                @pl.when(o < NRND - 1)
                def _():
                    _idx_fetch(ei, padc, ib.at[b - 1],
                               qbase + o * NIB + b + 5, isem)
        return c

    lax.fori_loop(0, NRND, round_body, 0)
    scat_wait(NIB - 1, (KE - 1) % NBUF)
    plsc.subcore_barrier()

    tile_io(True)


_edge = pl.kernel(
    _edge_body,
    out_type=jax.ShapeDtypeStruct((NC, NR, D), jnp.float32),
    mesh=plsc.VectorSubcoreMesh(core_axis_name="c", subcore_axis_name="s"),
    scratch_types=[
        pltpu.VMEM_SHARED((NR, D), jnp.float32),
        pltpu.VMEM((NIB, 2, CHUNK), jnp.int32),
        pltpu.VMEM((NBUF, CHUNK, D), jnp.float32),
    ] + [pltpu.SemaphoreType.DMA] * 7,
)


def _dis_from_counts(cnt_ref):
    c = cnt_ref[0, :, 0:1] + cnt_ref[1, :, 0:1] + 1.0
    return lax.rsqrt(c)


def _mm1_body(x_ref, w_ref, o_ref):
    o_ref[0:N, :] = jnp.dot(x_ref[...], w_ref[...],
                            preferred_element_type=jnp.float32)
    o_ref[N:NP, :] = jnp.zeros((NP - N, D), jnp.float32)


def _sc1_body(u_ref, cnt_ref, o_ref):
    o_ref[...] = u_ref[...] * _dis_from_counts(cnt_ref)


def _tc2_body(acc_ref, s_ref, cnt_ref, b_ref, w_ref, o_ref):
    dis = _dis_from_counts(cnt_ref)
    g = acc_ref[0, 0:N, :] + acc_ref[1, 0:N, :] + s_ref[0:N, :]
    gp = jnp.concatenate(
        [g, jnp.zeros((NP - N, D), jnp.float32)], axis=0)
    h = jnp.maximum(gp * dis + b_ref[...], 0.0)
    o_ref[...] = jnp.dot(h, w_ref[...],
                         preferred_element_type=jnp.float32) * dis


def _tc3_body(acc_ref, s_ref, cnt_ref, b_ref, o_ref):
    c = cnt_ref[0, 0:N, 0:1] + cnt_ref[1, 0:N, 0:1] + 1.0
    dis = lax.rsqrt(c)
    g = acc_ref[0, 0:N, :] + acc_ref[1, 0:N, :] + s_ref[0:N, :]
    o_ref[...] = g * dis + b_ref[...]


_mm1 = pl.pallas_call(
    _mm1_body, out_shape=jax.ShapeDtypeStruct((NP, D), jnp.float32))
_sc1 = pl.pallas_call(
    _sc1_body, out_shape=jax.ShapeDtypeStruct((NP, D), jnp.float32))
_tc2 = pl.pallas_call(
    _tc2_body, out_shape=jax.ShapeDtypeStruct((NP, D), jnp.float32))
_tc3 = pl.pallas_call(
    _tc3_body, out_shape=jax.ShapeDtypeStruct((N, D), jnp.float32))


def kernel(x, edge_index, W1, b1, W2, b2):
    ei = edge_index.astype(jnp.int32)
    padr = (N + np.arange(NCHP * CHUNK, dtype=np.int32) % (NR - N))
    padc = jnp.asarray(np.broadcast_to(padr.reshape(NCHP, 1, CHUNK),
                                       (NCHP, 2, CHUNK)))
    b1r = b1.reshape(1, D)
    b2r = b2.reshape(1, D)

    cnt = _deg(ei, padc)
    u1 = _mm1(x, W1)
    s1 = _sc1(u1, cnt)
    a1 = _edge(s1, ei, padc)
    s2 = _tc2(a1, s1, cnt, b1r, W2)
    a2 = _edge(s2, ei, padc)
    return _tc3(a2, s2, cnt, b2r)
